# baseline (speedup 1.0000x reference)
import jax
import jax.numpy as jnp
from jax import lax
from jax.experimental import pallas as pl
from jax.experimental.pallas import tpu as pltpu
from jax.experimental.pallas import tpu_sc as plsc

VOCAB = 100000
HIDDEN = 768
SEG_NUM = 3
B, L = 1024, 200
N = B * L
NC, NS, LANES = 2, 16, 16
NW = NC * NS
PER_W = N // NW
CHUNK = 32
NCHUNK = PER_W // CHUNK
HALF = NCHUNK // 2
G = HIDDEN // LANES
VBLK = 1000


def _combine_body(t_ref, p_ref, o_ref):
    o_ref[...] = t_ref[...] + p_ref[...]


def _combine(token_table, pos_table):
    spec = pl.BlockSpec((VBLK, HIDDEN), lambda i: (i, 0))
    return pl.pallas_call(
        _combine_body,
        grid=(VOCAB // VBLK,),
        in_specs=[spec, spec],
        out_specs=spec,
        out_shape=jax.ShapeDtypeStruct((VOCAB, HIDDEN), jnp.float32),
    )(token_table, pos_table)


def _body(token_hbm, seg_hbm, comb_tab, seg_tab_hbm, out_hbm,
          idx_all, seg_all, cb0, cb1, sb0, sb1, segtab_v,
          sc0, sc1, ss0, ss1, so0, so1):
    wid = lax.axis_index("s") * NC + lax.axis_index("c")
    base = wid * PER_W
    cbufs, sbufs = (cb0, cb1), (sb0, sb1)
    csems, ssems, osems = (sc0, sc1), (ss0, ss1), (so0, so1)

    pltpu.sync_copy(token_hbm.at[wid], idx_all)
    pltpu.sync_copy(seg_hbm.at[wid], seg_all)

    @pl.when(lax.axis_index("s") == 0)
    def _():
        pltpu.sync_copy(seg_tab_hbm, segtab_v)
    plsc.subcore_barrier()

    def idx_ref(g):
        return idx_all.at[pl.ds(g * CHUNK, CHUNK)]

    def issue_gathers(p, g):
        pltpu.async_copy(comb_tab.at[idx_ref(g)], cbufs[p], csems[p])
        for b in range(CHUNK // LANES):
            sv = seg_all[pl.ds(g * CHUNK + b * LANES, LANES)]
            for k in range(LANES):
                pltpu.async_copy(
                    segtab_v.at[pl.ds(sv[k] * HIDDEN, HIDDEN)],
                    sbufs[p].at[pl.ds((b * LANES + k) * HIDDEN, HIDDEN)],
                    ssems[p])

    def wait_gathers(p, g):
        pltpu.make_async_copy(comb_tab.at[idx_ref(g)], cbufs[p],
                              csems[p]).wait()
        for k in range(CHUNK):
            pltpu.make_async_copy(
                segtab_v.at[pl.ds(0, HIDDEN)],
                sbufs[p].at[pl.ds(k * HIDDEN, HIDDEN)], ssems[p]).wait()

    def issue_out(p, g):
        off = base + g * CHUNK
        pltpu.async_copy(cbufs[p], out_hbm.at[pl.ds(off, CHUNK)], osems[p])

    def wait_out(p):
        pltpu.make_async_copy(cbufs[p], out_hbm.at[pl.ds(base, CHUNK)],
                              osems[p]).wait()

    def compute(p):
        cb, sb = cbufs[p], sbufs[p]

        @plsc.parallel_loop(0, CHUNK, unroll=4)
        def _(r):
            for j in range(G):
                sl = pl.ds(j * LANES, LANES)
                ssl = pl.ds(r * HIDDEN + j * LANES, LANES)
                cb[r, sl] = cb[r, sl] + sb[ssl]

    issue_gathers(0, 0)

    def loop_body(gg, carry):
        g0 = 2 * gg
        g1 = g0 + 1

        @pl.when(gg > 0)
        def _():
            wait_out(1)
        issue_gathers(1, g1)
        wait_gathers(0, g0)
        compute(0)
        issue_out(0, g0)

        @pl.when(gg < HALF - 1)
        def _():
            wait_out(0)
            issue_gathers(0, g1 + 1)
        wait_gathers(1, g1)
        compute(1)
        issue_out(1, g1)
        return carry

    lax.fori_loop(0, HALF, loop_body, 0)
    wait_out(0)
    wait_out(1)


def kernel(token, segment, token_table, pos_table, seg_table):
    comb = _combine(token_table, pos_table)
    tok_r = token.reshape(NW, PER_W).astype(jnp.int32)
    seg_r = segment.reshape(NW, PER_W).astype(jnp.int32)
    mesh = plsc.VectorSubcoreMesh(core_axis_name="c", subcore_axis_name="s")
    out = pl.kernel(
        _body,
        mesh=mesh,
        out_type=jax.ShapeDtypeStruct((N, HIDDEN), jnp.float32),
        scratch_types=[
            pltpu.VMEM((PER_W,), jnp.int32),
            pltpu.VMEM((PER_W,), jnp.int32),
            pltpu.VMEM((CHUNK, HIDDEN), jnp.float32),
            pltpu.VMEM((CHUNK, HIDDEN), jnp.float32),
            pltpu.VMEM((CHUNK * HIDDEN,), jnp.float32),
            pltpu.VMEM((CHUNK * HIDDEN,), jnp.float32),
            pltpu.VMEM_SHARED((SEG_NUM * HIDDEN,), jnp.float32),
            pltpu.SemaphoreType.DMA,
            pltpu.SemaphoreType.DMA,
            pltpu.SemaphoreType.DMA,
            pltpu.SemaphoreType.DMA,
            pltpu.SemaphoreType.DMA,
            pltpu.SemaphoreType.DMA,
        ],
    )(tok_r, seg_r, comb, seg_table.reshape(-1))
    return out.reshape(B, L, HIDDEN)


# speedup vs baseline: 1.1473x; 1.1473x over previous
import jax
import jax.numpy as jnp
from jax import lax
from jax.experimental import pallas as pl
from jax.experimental.pallas import tpu as pltpu
from jax.experimental.pallas import tpu_sc as plsc

VOCAB = 100000
HIDDEN = 768
SEG_NUM = 3
B, L = 1024, 200
N = B * L
NC, NS, LANES = 2, 16, 16
NW = NC * NS
PER_W = N // NW
CHUNK = 32
NCHUNK = PER_W // CHUNK
HALF = NCHUNK // 2
G = HIDDEN // LANES
VBLK = 1000


def _combine_body(t_ref, p_ref, o_ref):
    o_ref[...] = t_ref[...] + p_ref[...]


def _combine(token_table, pos_table):
    spec = pl.BlockSpec((VBLK, HIDDEN), lambda i: (i, 0))
    return pl.pallas_call(
        _combine_body,
        grid=(VOCAB // VBLK,),
        in_specs=[spec, spec],
        out_specs=spec,
        out_shape=jax.ShapeDtypeStruct((VOCAB, HIDDEN), jnp.float32),
    )(token_table, pos_table)


def _body(token_hbm, seg_hbm, comb_tab, seg_tab_hbm, out_hbm,
          idx_all, seg_all, cb0, cb1, sb0, sb1, segtab_v,
          sc0, sc1, ss0, ss1, so0, so1):
    wid = lax.axis_index("s") * NC + lax.axis_index("c")
    base = wid * PER_W
    cbufs, sbufs = (cb0, cb1), (sb0, sb1)
    csems, ssems, osems = (sc0, sc1), (ss0, ss1), (so0, so1)

    pltpu.sync_copy(token_hbm.at[wid], idx_all)
    pltpu.sync_copy(seg_hbm.at[wid], seg_all)

    @pl.when(lax.axis_index("s") == 0)
    def _():
        pltpu.sync_copy(seg_tab_hbm, segtab_v)
    plsc.subcore_barrier()

    def idx_ref(g):
        return idx_all.at[pl.ds(g * CHUNK, CHUNK)]

    def issue_gathers(p, g):
        pltpu.async_copy(comb_tab.at[idx_ref(g)], cbufs[p], csems[p])
        for b in range(CHUNK // LANES):
            sv = seg_all[pl.ds(g * CHUNK + b * LANES, LANES)]
            for k in range(LANES):
                pltpu.async_copy(
                    segtab_v.at[pl.ds(sv[k] * HIDDEN, HIDDEN)],
                    sbufs[p].at[pl.ds((b * LANES + k) * HIDDEN, HIDDEN)],
                    ssems[p])

    def wait_gathers(p, g):
        pltpu.make_async_copy(comb_tab.at[idx_ref(g)], cbufs[p],
                              csems[p]).wait()
        for k in range(CHUNK):
            pltpu.make_async_copy(
                segtab_v.at[pl.ds(0, HIDDEN)],
                sbufs[p].at[pl.ds(k * HIDDEN, HIDDEN)], ssems[p]).wait()

    def issue_out(p, g):
        off = base + g * CHUNK
        pltpu.async_copy(cbufs[p], out_hbm.at[pl.ds(off, CHUNK)], osems[p])

    def wait_out(p):
        pltpu.make_async_copy(cbufs[p], out_hbm.at[pl.ds(base, CHUNK)],
                              osems[p]).wait()

    def compute(p):
        cb, sb = cbufs[p], sbufs[p]

        @plsc.parallel_loop(0, CHUNK, unroll=1)
        def _(r):
            for j in range(G):
                sl = pl.ds(j * LANES, LANES)
                ssl = pl.ds(r * HIDDEN + j * LANES, LANES)
                cb[r, sl] = cb[r, sl] + sb[ssl]

    issue_gathers(0, 0)

    def loop_body(gg, carry):
        g0 = 2 * gg
        g1 = g0 + 1

        @pl.when(gg > 0)
        def _():
            wait_out(1)
        issue_gathers(1, g1)
        wait_gathers(0, g0)
        compute(0)
        issue_out(0, g0)

        @pl.when(gg < HALF - 1)
        def _():
            wait_out(0)
            issue_gathers(0, g1 + 1)
        wait_gathers(1, g1)
        compute(1)
        issue_out(1, g1)
        return carry

    lax.fori_loop(0, HALF, loop_body, 0)
    wait_out(0)
    wait_out(1)


def kernel(token, segment, token_table, pos_table, seg_table):
    comb = _combine(token_table, pos_table)
    tok_r = token.reshape(NW, PER_W).astype(jnp.int32)
    seg_r = segment.reshape(NW, PER_W).astype(jnp.int32)
    mesh = plsc.VectorSubcoreMesh(core_axis_name="c", subcore_axis_name="s")
    out = pl.kernel(
        _body,
        mesh=mesh,
        out_type=jax.ShapeDtypeStruct((N, HIDDEN), jnp.float32),
        scratch_types=[
            pltpu.VMEM((PER_W,), jnp.int32),
            pltpu.VMEM((PER_W,), jnp.int32),
            pltpu.VMEM((CHUNK, HIDDEN), jnp.float32),
            pltpu.VMEM((CHUNK, HIDDEN), jnp.float32),
            pltpu.VMEM((CHUNK * HIDDEN,), jnp.float32),
            pltpu.VMEM((CHUNK * HIDDEN,), jnp.float32),
            pltpu.VMEM_SHARED((SEG_NUM * HIDDEN,), jnp.float32),
            pltpu.SemaphoreType.DMA,
            pltpu.SemaphoreType.DMA,
            pltpu.SemaphoreType.DMA,
            pltpu.SemaphoreType.DMA,
            pltpu.SemaphoreType.DMA,
            pltpu.SemaphoreType.DMA,
        ],
    )(tok_r, seg_r, comb, seg_table.reshape(-1))
    return out.reshape(B, L, HIDDEN)
